# baseline (speedup 1.0000x reference)
import functools

import jax
import jax.numpy as jnp
from jax import lax
from jax.experimental import pallas as pl
from jax.experimental.pallas import tpu as pltpu
from jax.experimental.pallas import tpu_sc as plsc

N = 10000
E = 160000
D = 128
DD = 64
NB = 32
K = 2
R = 4

NC = 2
NS = 16
NW = NC * NS
L = 16

NP = 10240
NPF = R * NP
NPS = NP // NS
DUMMY = N
CH = 128
NCHUNK = 40
ET = NCHUNK * CH
EP = NW * ET
TW = 40

CHR = 64
NCHR = ET // CHR

BM = 1024

_params = pltpu.CompilerParams(needs_layout_passes=False)
_mesh = plsc.VectorSubcoreMesh(core_axis_name="c", subcore_axis_name="s",
                               num_cores=NC, num_subcores=NS)
_ES = (NW, NCHUNK, CH)


def _zero1d(ref, n):
    def body(i, carry):
        ref[pl.ds(L * i, L)] = jnp.zeros((L,), jnp.float32)
        return carry

    lax.fori_loop(0, n // L, body, 0)


@functools.partial(
    pl.kernel,
    out_type=jax.ShapeDtypeStruct((NW, 2, NP), jnp.float32),
    mesh=_mesh,
    compiler_params=_params,
    scratch_types=[
        pltpu.VMEM((NCHUNK, CH), jnp.int32),
        pltpu.VMEM((NCHUNK, CH), jnp.int32),
        pltpu.VMEM((NP,), jnp.float32),
        pltpu.VMEM((NP,), jnp.float32),
    ],
)
def _sc_degrees(dst_hbm, src_hbm, out_hbm, dstv, srcv, dga, dgb):
    c = lax.axis_index("c")
    s = lax.axis_index("s")
    wid = c * NS + s
    pltpu.sync_copy(dst_hbm.at[wid], dstv)
    pltpu.sync_copy(src_hbm.at[wid], srcv)
    _zero1d(dga, NP)
    _zero1d(dgb, NP)

    def chunk(g, carry):
        def grp(i, carry2):
            sl = pl.ds(L * i, L)
            one = jnp.full((L,), 1.0, jnp.float32)
            plsc.addupdate_scatter(dga, [dstv[g, sl]], one)
            plsc.addupdate_scatter(dgb, [srcv[g, sl]], one)
            return carry2

        lax.fori_loop(0, CH // L, grp, 0)
        return carry

    lax.fori_loop(0, NCHUNK, chunk, 0)
    pltpu.sync_copy(dga, out_hbm.at[wid].at[0])
    pltpu.sync_copy(dgb, out_hbm.at[wid].at[1])


@functools.partial(
    pl.kernel,
    out_type=(
        jax.ShapeDtypeStruct(_ES, jnp.float32),
        jax.ShapeDtypeStruct(_ES, jnp.float32),
    ),
    mesh=_mesh,
    compiler_params=_params,
    scratch_types=[
        pltpu.VMEM((NCHUNK, CH), jnp.int32),
        pltpu.VMEM((NCHUNK, CH), jnp.int32),
        pltpu.VMEM((NCHUNK, CH), jnp.int32),
        pltpu.VMEM((NPF,), jnp.float32),
        pltpu.VMEM((NPF,), jnp.float32),
        pltpu.VMEM((NP,), jnp.float32),
        pltpu.VMEM((NCHUNK, CH), jnp.float32),
        pltpu.VMEM((NCHUNK, CH), jnp.float32),
    ],
)
def _sc_gate_a(dst_hbm, mid_hbm, et_hbm, s1_hbm, s2_hbm, d0_hbm,
               part_hbm, d0e_hbm,
               dstv, midv, etv, s1v, s2v, d0v, partv, d0ev):
    c = lax.axis_index("c")
    s = lax.axis_index("s")
    wid = c * NS + s
    pltpu.sync_copy(dst_hbm.at[wid], dstv)
    pltpu.sync_copy(mid_hbm.at[wid], midv)
    pltpu.sync_copy(et_hbm.at[wid], etv)
    pltpu.sync_copy(s1_hbm, s1v)
    pltpu.sync_copy(s2_hbm, s2v)
    pltpu.sync_copy(d0_hbm, d0v)

    def chunk(g, carry):
        def grp(i, carry2):
            sl = pl.ds(L * i, L)
            d16 = dstv[g, sl]
            m16 = midv[g, sl]
            base = etv[g, sl] * NP
            s1g = plsc.load_gather(s1v, [base + d16])
            s2g = plsc.load_gather(s2v, [base + m16])
            d0g = plsc.load_gather(d0v, [d16])
            partv[g, sl] = s1g + s2g
            d0ev[g, sl] = d0g
            return carry2

        lax.fori_loop(0, CH // L, grp, 0)
        return carry

    lax.fori_loop(0, NCHUNK, chunk, 0)
    pltpu.sync_copy(partv, part_hbm.at[wid])
    pltpu.sync_copy(d0ev, d0e_hbm.at[wid])


@functools.partial(
    pl.kernel,
    out_type=(
        jax.ShapeDtypeStruct(_ES, jnp.float32),
        jax.ShapeDtypeStruct(_ES, jnp.float32),
        jax.ShapeDtypeStruct(_ES, jnp.int32),
    ),
    mesh=_mesh,
    compiler_params=_params,
    scratch_types=[
        pltpu.VMEM((NCHUNK, CH), jnp.int32),
        pltpu.VMEM((NCHUNK, CH), jnp.int32),
        pltpu.VMEM((NCHUNK, CH), jnp.int32),
        pltpu.VMEM((NCHUNK, CH), jnp.float32),
        pltpu.VMEM((NCHUNK, CH), jnp.float32),
        pltpu.VMEM((NPF,), jnp.float32),
        pltpu.VMEM((NP,), jnp.float32),
        pltpu.VMEM((NP,), jnp.float32),
        pltpu.VMEM((NP,), jnp.float32),
        pltpu.VMEM((R * TW,), jnp.float32),
        pltpu.VMEM((NB,), jnp.float32),
        pltpu.VMEM((NCHUNK, CH), jnp.float32),
        pltpu.VMEM((NCHUNK, CH), jnp.float32),
        pltpu.VMEM((NCHUNK, CH), jnp.int32),
    ],
)
def _sc_gate_b(src_hbm, dst_hbm, et_hbm, part_hbm, d0e_hbm, s3_hbm, d2_hbm,
               lx_hbm, ly_hbm, t_hbm, b2_hbm,
               c1_hbm, sc_hbm, bi_hbm,
               srcv, dstv, etv, partv, d0ev, s3v, d2v, lxv, lyv, tv, b2v,
               c1v, scv, biv):
    c = lax.axis_index("c")
    s = lax.axis_index("s")
    wid = c * NS + s
    pltpu.sync_copy(src_hbm.at[wid], srcv)
    pltpu.sync_copy(dst_hbm.at[wid], dstv)
    pltpu.sync_copy(et_hbm.at[wid], etv)
    pltpu.sync_copy(part_hbm.at[wid], partv)
    pltpu.sync_copy(d0e_hbm.at[wid], d0ev)
    pltpu.sync_copy(s3_hbm, s3v)
    pltpu.sync_copy(d2_hbm, d2v)
    pltpu.sync_copy(lx_hbm, lxv)
    pltpu.sync_copy(ly_hbm, lyv)
    pltpu.sync_copy(t_hbm, tv)
    pltpu.sync_copy(b2_hbm, b2v)

    b2lo = b2v[pl.ds(0, L)]
    b2hi = b2v[pl.ds(L, L)]
    b2s = [b2lo[j] for j in range(L)] + [b2hi[j] for j in range(L)]

    def chunk(g, carry):
        def grp(i, carry2):
            sl = pl.ds(L * i, L)
            s16 = srcv[g, sl]
            d16 = dstv[g, sl]
            e16 = etv[g, sl]
            lxd = plsc.load_gather(lxv, [d16])
            lyd = plsc.load_gather(lyv, [d16])
            lxs = plsc.load_gather(lxv, [s16])
            lys = plsc.load_gather(lyv, [s16])
            dx = lxd - lxs
            dy = lyd - lys
            q = dx * dx + dy * dy + jnp.float32(1e-12)
            cnt = jnp.zeros((L,), jnp.int32)
            for j in range(NB):
                cnt = cnt + jnp.where(b2s[j] < q, 1, 0).astype(jnp.int32)
            tg = plsc.load_gather(tv, [e16 * TW + cnt])
            s3g = plsc.load_gather(s3v, [e16 * NP + s16])
            d2g = plsc.load_gather(d2v, [s16])
            x = partv[g, sl] + tg + s3g
            beta = 1.0 / (1.0 + jnp.exp(-x))
            scl = d0ev[g, sl] * d2g
            biv[g, sl] = cnt
            scv[g, sl] = scl
            c1v[g, sl] = scl * beta
            return carry2

        lax.fori_loop(0, CH // L, grp, 0)
        return carry

    lax.fori_loop(0, NCHUNK, chunk, 0)
    pltpu.sync_copy(c1v, c1_hbm.at[wid])
    pltpu.sync_copy(scv, sc_hbm.at[wid])
    pltpu.sync_copy(biv, bi_hbm.at[wid])


@functools.partial(
    pl.kernel,
    out_type=(
        jax.ShapeDtypeStruct(_ES, jnp.int32),
        jax.ShapeDtypeStruct(_ES, jnp.int32),
        jax.ShapeDtypeStruct(_ES, jnp.int32),
        jax.ShapeDtypeStruct(_ES, jnp.int32),
    ),
    mesh=_mesh,
    compiler_params=_params,
    scratch_types=[
        pltpu.VMEM((NCHUNK, CH), jnp.int32),
        pltpu.VMEM((NCHUNK, CH), jnp.int32),
        pltpu.VMEM((NCHUNK, CH), jnp.int32),
        pltpu.VMEM((NCHUNK, CH), jnp.int32),
        pltpu.VMEM((NCHUNK, CH), jnp.int32),
        pltpu.VMEM((NP,), jnp.float32),
        pltpu.VMEM((NP,), jnp.float32),
        pltpu.VMEM((NB,), jnp.float32),
        pltpu.VMEM((NCHUNK, CH), jnp.int32),
        pltpu.VMEM((NCHUNK, CH), jnp.int32),
        pltpu.VMEM((NCHUNK, CH), jnp.int32),
        pltpu.VMEM((NCHUNK, CH), jnp.int32),
    ],
)
def _sc_gate_c(src_hbm, mid_hbm, et_hbm, i0_hbm, i1_hbm, lx_hbm, ly_hbm,
               b2_hbm,
               gs_hbm, gm_hbm, b0_hbm, b1_hbm,
               srcv, midv, etv, i0v, i1v, lxv, lyv, b2v,
               gsv, gmv, b0v, b1v):
    c = lax.axis_index("c")
    s = lax.axis_index("s")
    wid = c * NS + s
    pltpu.sync_copy(src_hbm.at[wid], srcv)
    pltpu.sync_copy(mid_hbm.at[wid], midv)
    pltpu.sync_copy(et_hbm.at[wid], etv)
    pltpu.sync_copy(i0_hbm.at[wid], i0v)
    pltpu.sync_copy(i1_hbm.at[wid], i1v)
    pltpu.sync_copy(lx_hbm, lxv)
    pltpu.sync_copy(ly_hbm, lyv)
    pltpu.sync_copy(b2_hbm, b2v)

    b2lo = b2v[pl.ds(0, L)]
    b2hi = b2v[pl.ds(L, L)]
    b2s = [b2lo[j] for j in range(L)] + [b2hi[j] for j in range(L)]

    def chunk(g, carry):
        def grp(i, carry2):
            sl = pl.ds(L * i, L)
            base = etv[g, sl] * NP
            s16 = srcv[g, sl]
            gsv[g, sl] = base + s16
            gmv[g, sl] = base + midv[g, sl]
            lxs = plsc.load_gather(lxv, [s16])
            lys = plsc.load_gather(lyv, [s16])
            for iv, bv in ((i0v, b0v), (i1v, b1v)):
                n16 = iv[g, sl]
                lx = plsc.load_gather(lxv, [n16])
                ly = plsc.load_gather(lyv, [n16])
                dx = lxs - lx
                dy = lys - ly
                q = dx * dx + dy * dy + jnp.float32(1e-12)
                cnt = jnp.zeros((L,), jnp.int32)
                for j in range(NB):
                    cnt = cnt + jnp.where(b2s[j] < q, 1, 0).astype(jnp.int32)
                bv[g, sl] = cnt
            return carry2

        lax.fori_loop(0, CH // L, grp, 0)
        return carry

    lax.fori_loop(0, NCHUNK, chunk, 0)
    pltpu.sync_copy(gsv, gs_hbm.at[wid])
    pltpu.sync_copy(gmv, gm_hbm.at[wid])
    pltpu.sync_copy(b0v, b0_hbm.at[wid])
    pltpu.sync_copy(b1v, b1_hbm.at[wid])


@functools.partial(
    pl.kernel,
    out_type=jax.ShapeDtypeStruct((NC, NP, D), jnp.float32),
    mesh=_mesh,
    compiler_params=_params,
    scratch_types=(
        [pltpu.VMEM((CHR,), jnp.int32)] * 3
        + [pltpu.VMEM((CHR,), jnp.float32)] * 2
        + [pltpu.VMEM((CHR, D), jnp.float32)] * 2
    ) * 2 + [
        pltpu.VMEM((CHR, D), jnp.float32),
        pltpu.VMEM_SHARED((NP, D), jnp.float32),
        pltpu.SemaphoreType.DMA,
        pltpu.SemaphoreType.DMA,
        pltpu.SemaphoreType.DMA,
        pltpu.SemaphoreType.DMA,
        pltpu.SemaphoreType.DMA,
        pltpu.SemaphoreType.DMA,
    ],
)
def _sc_combine1(gs_hbm, gm_hbm, dst_hbm, c1_hbm, sc_hbm, h2_hbm,
                 zD_hbm, hp_hbm,
                 gsv0, gmv0, dstv0, c1v0, scv0, hsrc0, hmid0,
                 gsv1, gmv1, dstv1, c1v1, scv1, hsrc1, hmid1,
                 outr, acc, sem0, sem1, semL0, semL1, semM0, semM1):
    c = lax.axis_index("c")
    s = lax.axis_index("s")
    wid = c * NS + s
    pltpu.sync_copy(zD_hbm.at[pl.ds(s * NPS, NPS)], acc.at[pl.ds(s * NPS, NPS)])
    plsc.subcore_barrier()

    sets = ((gsv0, gmv0, dstv0, c1v0, scv0, hsrc0, hmid0, sem0, semL0, semM0),
            (gsv1, gmv1, dstv1, c1v1, scv1, hsrc1, hmid1, sem1, semL1, semM1))

    def _ld(g, st, op):
        gsv, gmv, dstv, c1v, scv = st[:5]
        sem = st[8]
        base = g * CHR
        op(gs_hbm.at[wid].at[pl.ds(base, CHR)], gsv, sem)
        op(gm_hbm.at[wid].at[pl.ds(base, CHR)], gmv, sem)
        op(dst_hbm.at[wid].at[pl.ds(base, CHR)], dstv, sem)
        op(c1_hbm.at[wid].at[pl.ds(base, CHR)], c1v, sem)
        op(sc_hbm.at[wid].at[pl.ds(base, CHR)], scv, sem)

    def fire_loads(g, st):
        _ld(g, st, pltpu.async_copy)

    def drain_loads(g, st):
        _ld(g, st, lambda a, b, sm: pltpu.make_async_copy(a, b, sm).wait())

    def fire(st):
        gsv, gmv = st[0], st[1]
        hsrc, hmid, sem, semm = st[5], st[6], st[7], st[9]
        pltpu.async_copy(h2_hbm.at[gsv], hsrc, sem)
        pltpu.async_copy(h2_hbm.at[gmv], hmid, semm)

    def drain(st):
        gsv, gmv = st[0], st[1]
        hsrc, hmid, sem, semm = st[5], st[6], st[7], st[9]
        pltpu.make_async_copy(h2_hbm.at[gsv], hsrc, sem).wait()
        pltpu.make_async_copy(h2_hbm.at[gmv], hmid, semm).wait()

    def work(g, st):
        dstv, c1v, scv, hsrc, hmid = st[2], st[3], st[4], st[5], st[6]

        def combine(i, carry2):
            sl = pl.ds(L * i, L)
            a16 = c1v[sl]
            b16 = scv[sl]
            for e in range(L):
                a = a16[e]
                b = b16[e]
                ee = L * i + e
                for v in range(D // L):
                    sv = pl.ds(L * v, L)
                    outr[ee, sv] = a * hsrc[ee, sv] + b * hmid[ee, sv]
            return carry2

        lax.fori_loop(0, CHR // L, combine, 0)
        pltpu.sync_copy(outr, acc.at[dstv], add=True)

    fire_loads(0, sets[0])
    drain_loads(0, sets[0])
    fire(sets[0])
    fire_loads(1, sets[1])

    def pair(p, carry):
        for b in (0, 1):
            g = 2 * p + b
            gn = jnp.minimum(g + 1, NCHR - 1)
            gnn = jnp.minimum(g + 2, NCHR - 1)
            drain_loads(gn, sets[1 - b])
            fire(sets[1 - b])
            drain(sets[b])
            work(g, sets[b])
            fire_loads(gnn, sets[b])
        return carry

    lax.fori_loop(0, NCHR // 2, pair, 0)
    drain(sets[0])
    drain_loads(0, sets[1])
    plsc.subcore_barrier()
    sl = pl.ds(s * NPS, NPS)
    pltpu.sync_copy(acc.at[sl], hp_hbm.at[c].at[sl])


@functools.partial(
    pl.kernel,
    out_type=(
        jax.ShapeDtypeStruct((NC, NP, D), jnp.float32),
        jax.ShapeDtypeStruct((NW, NP), jnp.float32),
    ),
    mesh=_mesh,
    compiler_params=_params,
    scratch_types=(
        [pltpu.VMEM((CHR,), jnp.int32)] * 3
        + [pltpu.VMEM((CHR,), jnp.float32)]
        + [pltpu.VMEM((CHR, D), jnp.float32)]
    ) * 2 + [
        pltpu.VMEM((TW, D), jnp.float32),
        pltpu.VMEM((NP,), jnp.float32),
        pltpu.VMEM((CHR, D), jnp.float32),
        pltpu.VMEM_SHARED((NP, D), jnp.float32),
        pltpu.SemaphoreType.DMA,
        pltpu.SemaphoreType.DMA,
        pltpu.SemaphoreType.DMA,
        pltpu.SemaphoreType.DMA,
    ],
)
def _sc_stage2a(src_hbm, dst_hbm, bi_hbm, sc_hbm, f2_hbm, tg_hbm, zD_hbm,
                s1p_hbm, scp_hbm,
                srcv0, dstv0, biv0, scv0, f2r0,
                srcv1, dstv1, biv1, scv1, f2r1,
                tgv, hist, outr, acc, sem0, sem1, semL0, semL1):
    c = lax.axis_index("c")
    s = lax.axis_index("s")
    wid = c * NS + s
    pltpu.sync_copy(tg_hbm, tgv)
    _zero1d(hist, NP)
    pltpu.sync_copy(zD_hbm.at[pl.ds(s * NPS, NPS)], acc.at[pl.ds(s * NPS, NPS)])
    plsc.subcore_barrier()

    sets = ((srcv0, dstv0, biv0, scv0, f2r0, sem0, semL0),
            (srcv1, dstv1, biv1, scv1, f2r1, sem1, semL1))

    def _ld(g, st, op):
        srcv, dstv, biv, scv = st[:4]
        sem = st[6]
        base = g * CHR
        op(src_hbm.at[wid].at[pl.ds(base, CHR)], srcv, sem)
        op(dst_hbm.at[wid].at[pl.ds(base, CHR)], dstv, sem)
        op(bi_hbm.at[wid].at[pl.ds(base, CHR)], biv, sem)
        op(sc_hbm.at[wid].at[pl.ds(base, CHR)], scv, sem)

    def fire_loads(g, st):
        _ld(g, st, pltpu.async_copy)

    def drain_loads(g, st):
        _ld(g, st, lambda a, b, sm: pltpu.make_async_copy(a, b, sm).wait())

    def fire(st):
        pltpu.async_copy(f2_hbm.at[st[0]], st[4], st[5])

    def drain(st):
        pltpu.make_async_copy(f2_hbm.at[st[0]], st[4], st[5]).wait()

    def work(g, st):
        dstv, biv, scv, f2r = st[1], st[2], st[3], st[4]

        def grp(i, carry2):
            sl = pl.ds(L * i, L)
            plsc.addupdate_scatter(hist, [dstv[sl]], scv[sl])
            b16 = biv[sl]
            s16 = scv[sl]
            for e in range(L):
                b = b16[e]
                scl = s16[e]
                ee = L * i + e
                for v in range(D // L):
                    sv = pl.ds(L * v, L)
                    outr[ee, sv] = scl * tgv[b, sv] * f2r[ee, sv]
            return carry2

        lax.fori_loop(0, CHR // L, grp, 0)
        pltpu.sync_copy(outr, acc.at[dstv], add=True)

    fire_loads(0, sets[0])
    drain_loads(0, sets[0])
    fire(sets[0])
    fire_loads(1, sets[1])

    def pair(p, carry):
        for b in (0, 1):
            g = 2 * p + b
            gn = jnp.minimum(g + 1, NCHR - 1)
            gnn = jnp.minimum(g + 2, NCHR - 1)
            drain_loads(gn, sets[1 - b])
            fire(sets[1 - b])
            drain(sets[b])
            work(g, sets[b])
            fire_loads(gnn, sets[b])
        return carry

    lax.fori_loop(0, NCHR // 2, pair, 0)
    drain(sets[0])
    drain_loads(0, sets[1])
    plsc.subcore_barrier()
    sl = pl.ds(s * NPS, NPS)
    pltpu.sync_copy(acc.at[sl], s1p_hbm.at[c].at[sl])
    pltpu.sync_copy(hist, scp_hbm.at[wid])


@functools.partial(
    pl.kernel,
    out_type=jax.ShapeDtypeStruct((NC, NP, D), jnp.float32),
    mesh=_mesh,
    compiler_params=_params,
    scratch_types=(
        [pltpu.VMEM((CHR,), jnp.int32)] * 5
        + [pltpu.VMEM((CHR,), jnp.float32)]
        + [pltpu.VMEM((CHR, D), jnp.float32)] * 2
    ) * 2 + [
        pltpu.VMEM((TW, D), jnp.float32),
        pltpu.VMEM((CHR, D), jnp.float32),
        pltpu.VMEM_SHARED((NP, D), jnp.float32),
        pltpu.SemaphoreType.DMA,
        pltpu.SemaphoreType.DMA,
        pltpu.SemaphoreType.DMA,
        pltpu.SemaphoreType.DMA,
        pltpu.SemaphoreType.DMA,
        pltpu.SemaphoreType.DMA,
    ],
)
def _sc_stage2b(i0_hbm, i1_hbm, dst_hbm, b0_hbm, b1_hbm, sc_hbm, f2_hbm,
                tg_hbm, zD_hbm,
                s2p_hbm,
                i0v0, i1v0, dstv0, b0v0, b1v0, scv0, f0r0, f1r0,
                i0v1, i1v1, dstv1, b0v1, b1v1, scv1, f0r1, f1r1,
                tgv, outr, acc, sem0, sem1, semL0, semL1, semM0, semM1):
    c = lax.axis_index("c")
    s = lax.axis_index("s")
    wid = c * NS + s
    pltpu.sync_copy(tg_hbm, tgv)
    pltpu.sync_copy(zD_hbm.at[pl.ds(s * NPS, NPS)], acc.at[pl.ds(s * NPS, NPS)])
    plsc.subcore_barrier()

    sets = ((i0v0, i1v0, dstv0, b0v0, b1v0, scv0, f0r0, f1r0, sem0, semL0, semM0),
            (i0v1, i1v1, dstv1, b0v1, b1v1, scv1, f0r1, f1r1, sem1, semL1, semM1))

    def _ld(g, st, op):
        i0v, i1v, dstv, b0v, b1v, scv = st[:6]
        sem = st[9]
        base = g * CHR
        op(i0_hbm.at[wid].at[pl.ds(base, CHR)], i0v, sem)
        op(i1_hbm.at[wid].at[pl.ds(base, CHR)], i1v, sem)
        op(dst_hbm.at[wid].at[pl.ds(base, CHR)], dstv, sem)
        op(b0_hbm.at[wid].at[pl.ds(base, CHR)], b0v, sem)
        op(b1_hbm.at[wid].at[pl.ds(base, CHR)], b1v, sem)
        op(sc_hbm.at[wid].at[pl.ds(base, CHR)], scv, sem)

    def fire_loads(g, st):
        _ld(g, st, pltpu.async_copy)

    def drain_loads(g, st):
        _ld(g, st, lambda a, b, sm: pltpu.make_async_copy(a, b, sm).wait())

    def fire(st):
        pltpu.async_copy(f2_hbm.at[st[0]], st[6], st[8])
        pltpu.async_copy(f2_hbm.at[st[1]], st[7], st[10])

    def drain(st):
        pltpu.make_async_copy(f2_hbm.at[st[0]], st[6], st[8]).wait()
        pltpu.make_async_copy(f2_hbm.at[st[1]], st[7], st[10]).wait()

    def work(g, st):
        dstv, b0v, b1v, scv, f0r, f1r = (st[2], st[3], st[4], st[5],
                                         st[6], st[7])

        def combine(i, carry2):
            sl = pl.ds(L * i, L)
            b016 = b0v[sl]
            b116 = b1v[sl]
            s16 = scv[sl]
            for e in range(L):
                b0 = b016[e]
                b1 = b116[e]
                scl = jnp.float32(0.5) * s16[e]
                ee = L * i + e
                for v in range(D // L):
                    sv = pl.ds(L * v, L)
                    outr[ee, sv] = scl * (tgv[b0, sv] * f0r[ee, sv]
                                          + tgv[b1, sv] * f1r[ee, sv])
            return carry2

        lax.fori_loop(0, CHR // L, combine, 0)
        pltpu.sync_copy(outr, acc.at[dstv], add=True)

    fire_loads(0, sets[0])
    drain_loads(0, sets[0])
    fire(sets[0])
    fire_loads(1, sets[1])

    def pair(p, carry):
        for b in (0, 1):
            g = 2 * p + b
            gn = jnp.minimum(g + 1, NCHR - 1)
            gnn = jnp.minimum(g + 2, NCHR - 1)
            drain_loads(gn, sets[1 - b])
            fire(sets[1 - b])
            drain(sets[b])
            work(g, sets[b])
            fire_loads(gnn, sets[b])
        return carry

    lax.fori_loop(0, NCHR // 2, pair, 0)
    drain(sets[0])
    drain_loads(0, sets[1])
    plsc.subcore_barrier()
    sl = pl.ds(s * NPS, NPS)
    pltpu.sync_copy(acc.at[sl], s2p_hbm.at[c].at[sl])


def _tc_prep_body(feat_ref, fc2_ref, a8_ref, degp_ref,
                  h2_ref, s1_ref, s2_ref, s3_ref, d0_ref, d2_ref):
    h2 = jnp.dot(feat_ref[...], fc2_ref[0], preferred_element_type=jnp.float32)
    h2_ref[...] = h2
    sc3 = jnp.dot(h2, a8_ref[0], preferred_element_type=jnp.float32)
    s1_ref[...] = sc3[:, 0].reshape(BM // D, D)
    s2_ref[...] = sc3[:, 1].reshape(BM // D, D)
    s3_ref[...] = sc3[:, 2].reshape(BM // D, D)
    degsum = jnp.sum(degp_ref[...], axis=0)
    d0_ref[...] = lax.rsqrt(jnp.maximum(degsum[0], 1.0)).reshape(BM // D, D)
    d2_ref[...] = lax.rsqrt(jnp.maximum(degsum[1], 1.0)).reshape(BM // D, D)


def _tc_prep(featp, fc2, a8, degp):
    nblk = NP // BM
    rb = BM // D
    return pl.pallas_call(
        _tc_prep_body,
        grid=(R, nblk),
        in_specs=[
            pl.BlockSpec((BM, D), lambda r, j: (j, 0)),
            pl.BlockSpec((1, D, D), lambda r, j: (r, 0, 0)),
            pl.BlockSpec((1, D, 8), lambda r, j: (r, 0, 0)),
            pl.BlockSpec((NW, 2, BM), lambda r, j: (0, 0, j)),
        ],
        out_specs=[
            pl.BlockSpec((BM, D), lambda r, j: (r * (NP // BM) + j, 0)),
            pl.BlockSpec((rb, D), lambda r, j: (r * (NP // BM) + j, 0)),
            pl.BlockSpec((rb, D), lambda r, j: (r * (NP // BM) + j, 0)),
            pl.BlockSpec((rb, D), lambda r, j: (r * (NP // BM) + j, 0)),
            pl.BlockSpec((rb, D), lambda r, j: (j, 0)),
            pl.BlockSpec((rb, D), lambda r, j: (j, 0)),
        ],
        out_shape=[
            jax.ShapeDtypeStruct((NPF, D), jnp.float32),
            jax.ShapeDtypeStruct((NPF // D, D), jnp.float32),
            jax.ShapeDtypeStruct((NPF // D, D), jnp.float32),
            jax.ShapeDtypeStruct((NPF // D, D), jnp.float32),
            jax.ShapeDtypeStruct((NP // D, D), jnp.float32),
            jax.ShapeDtypeStruct((NP // D, D), jnp.float32),
        ],
    )(featp, fc2, a8, degp)


def _tc_mid_body(hp_ref, fcp_ref, f2_ref):
    h = jnp.tanh(hp_ref[0] + hp_ref[1])
    f2_ref[...] = jnp.dot(h, fcp_ref[...], preferred_element_type=jnp.float32)


def _tc_mid(hpart, fc_p):
    return pl.pallas_call(
        _tc_mid_body,
        grid=(NP // BM,),
        in_specs=[
            pl.BlockSpec((NC, BM, D), lambda j: (0, j, 0)),
            pl.BlockSpec((D, D), lambda j: (0, 0)),
        ],
        out_specs=pl.BlockSpec((BM, D), lambda j: (j, 0)),
        out_shape=jax.ShapeDtypeStruct((NP, D), jnp.float32),
    )(hpart, fc_p)


def _tc_final_body(s1p_ref, s2p_ref, scp_ref, wa_ref, wb_ref, bb_ref, out_ref):
    s1 = s1p_ref[0] + s1p_ref[1]
    s2 = s2p_ref[0] + s2p_ref[1]
    sc = jnp.sum(scp_ref[...], axis=0)
    rst = (jnp.dot(s1, wa_ref[...], preferred_element_type=jnp.float32)
           + jnp.dot(s2, wb_ref[...], preferred_element_type=jnp.float32)
           + sc[:, None] * bb_ref[0][None, :])
    out_ref[...] = jnp.tanh(rst)


def _tc_final(s1p, s2p, scp, wa, wb, bb):
    return pl.pallas_call(
        _tc_final_body,
        grid=(NP // BM,),
        in_specs=[
            pl.BlockSpec((NC, BM, D), lambda j: (0, j, 0)),
            pl.BlockSpec((NC, BM, D), lambda j: (0, j, 0)),
            pl.BlockSpec((NW, BM), lambda j: (0, j)),
            pl.BlockSpec((D, D), lambda j: (0, 0)),
            pl.BlockSpec((D, D), lambda j: (0, 0)),
            pl.BlockSpec((1, D), lambda j: (0, 0)),
        ],
        out_specs=pl.BlockSpec((BM, D), lambda j: (j, 0)),
        out_shape=jax.ShapeDtypeStruct((NP, D), jnp.float32),
    )(s1p, s2p, scp, wa, wb, bb)


def kernel(feat, loc, src, dst, mid, inter_ids, etype, boundaries, embed_table,
           fc2, fcd, fc_w1, fc_w2, vec_a, fc_p, agg_W, agg_b, G):
    f32 = jnp.float32
    i32 = jnp.int32

    featp = jnp.zeros((NP, D), f32).at[:N].set(feat)
    locx = jnp.zeros((NP,), f32).at[:N].set(loc[:, 0])
    locy = jnp.zeros((NP,), f32).at[:N].set(loc[:, 1])

    pad = EP - E

    def pad_idx(x, val):
        return jnp.concatenate(
            [x.astype(i32), jnp.full((pad,), val, i32)]).reshape(NW, NCHUNK, CH)

    src_r = pad_idx(src, DUMMY)
    dst_r = pad_idx(dst, DUMMY)
    mid_r = pad_idx(mid, DUMMY)
    et_r = pad_idx(etype, 0)
    i0_r = pad_idx(inter_ids[:, 0], DUMMY)
    i1_r = pad_idx(inter_ids[:, 1], DUMMY)

    b2 = (boundaries * boundaries).astype(f32)

    a1 = jnp.einsum('rdk,rko->rd', fc_w1[:, :D, :], vec_a)
    a2 = jnp.einsum('rdk,rko->rd', fc_w2[:, :D, :], vec_a)
    a3 = jnp.einsum('rdk,rko->rd', fc_w2[:, D:, :], vec_a)
    a8 = jnp.concatenate(
        [a1[..., None], a2[..., None], a3[..., None],
         jnp.zeros((R, D, 5), f32)], axis=-1)
    td = jnp.einsum('rdk,rko->rd', fc_w1[:, D:, :], vec_a)
    tvec = jnp.einsum('red,rd->re', fcd, td)
    tbl = jnp.einsum('be,re->rb', embed_table, tvec)
    tflat = jnp.zeros((R, TW), f32).at[:, :NB + 1].set(tbl).reshape(-1)
    tgp = jnp.zeros((TW, D), f32).at[:NB + 1].set(embed_table @ G)

    zD = jnp.zeros((NP, D), f32)

    degp = _sc_degrees(dst_r, src_r)
    h2flat, s1t, s2t, s3t, d0t, d2t = _tc_prep(featp, fc2, a8, degp)
    s1f = s1t.reshape(-1)
    s2f = s2t.reshape(-1)
    s3f = s3t.reshape(-1)
    d0f = d0t.reshape(-1)
    d2f = d2t.reshape(-1)
    part, d0e = _sc_gate_a(dst_r, mid_r, et_r, s1f, s2f, d0f)
    c1_r, sc_r, bi_r = _sc_gate_b(src_r, dst_r, et_r, part, d0e, s3f, d2f,
                                  locx, locy, tflat, b2)
    gs_r, gm_r, b0_r, b1_r = _sc_gate_c(src_r, mid_r, et_r, i0_r, i1_r,
                                        locx, locy, b2)

    def rr(x):
        return x.reshape(NW, ET)

    hpart = _sc_combine1(rr(gs_r), rr(gm_r), rr(dst_r), rr(c1_r), rr(sc_r),
                         h2flat, zD)
    f2 = _tc_mid(hpart, fc_p)
    s1p, scp = _sc_stage2a(rr(src_r), rr(dst_r), rr(bi_r), rr(sc_r), f2, tgp,
                           zD)
    s2p = _sc_stage2b(rr(i0_r), rr(i1_r), rr(dst_r), rr(b0_r), rr(b1_r),
                      rr(sc_r), f2, tgp, zD)
    outp = _tc_final(s1p, s2p, scp, agg_W[:D], agg_W[D:],
                     agg_b.reshape(1, D).astype(f32))
    return outp[:N]


# speedup vs baseline: 1.0099x; 1.0099x over previous
import functools

import jax
import jax.numpy as jnp
from jax import lax
from jax.experimental import pallas as pl
from jax.experimental.pallas import tpu as pltpu
from jax.experimental.pallas import tpu_sc as plsc

N = 10000
E = 160000
D = 128
DD = 64
NB = 32
K = 2
R = 4

NC = 2
NS = 16
NW = NC * NS
L = 16

NP = 10240
NPF = R * NP
NPS = NP // NS
DUMMY = N
CH = 128
NCHUNK = 40
ET = NCHUNK * CH
EP = NW * ET
TW = 40

CHR = 64
NCHR = ET // CHR

BM = 1024

_params = pltpu.CompilerParams(needs_layout_passes=False)
_mesh = plsc.VectorSubcoreMesh(core_axis_name="c", subcore_axis_name="s",
                               num_cores=NC, num_subcores=NS)
_ES = (NW, NCHUNK, CH)


def _zero_rows(ref):
    def body(i, carry):
        for v in range(D // L):
            ref[i, pl.ds(L * v, L)] = jnp.zeros((L,), jnp.float32)
        return carry

    lax.fori_loop(0, CHR, body, 0)


def _zero1d(ref, n):
    def body(i, carry):
        ref[pl.ds(L * i, L)] = jnp.zeros((L,), jnp.float32)
        return carry

    lax.fori_loop(0, n // L, body, 0)


@functools.partial(
    pl.kernel,
    out_type=jax.ShapeDtypeStruct((NW, 2, NP), jnp.float32),
    mesh=_mesh,
    compiler_params=_params,
    scratch_types=[
        pltpu.VMEM((NCHUNK, CH), jnp.int32),
        pltpu.VMEM((NCHUNK, CH), jnp.int32),
        pltpu.VMEM((NP,), jnp.float32),
        pltpu.VMEM((NP,), jnp.float32),
    ],
)
def _sc_degrees(dst_hbm, src_hbm, out_hbm, dstv, srcv, dga, dgb):
    c = lax.axis_index("c")
    s = lax.axis_index("s")
    wid = c * NS + s
    pltpu.sync_copy(dst_hbm.at[wid], dstv)
    pltpu.sync_copy(src_hbm.at[wid], srcv)
    _zero1d(dga, NP)
    _zero1d(dgb, NP)

    def chunk(g, carry):
        def grp(i, carry2):
            sl = pl.ds(L * i, L)
            one = jnp.full((L,), 1.0, jnp.float32)
            plsc.addupdate_scatter(dga, [dstv[g, sl]], one)
            plsc.addupdate_scatter(dgb, [srcv[g, sl]], one)
            return carry2

        lax.fori_loop(0, CH // L, grp, 0)
        return carry

    lax.fori_loop(0, NCHUNK, chunk, 0)
    pltpu.sync_copy(dga, out_hbm.at[wid].at[0])
    pltpu.sync_copy(dgb, out_hbm.at[wid].at[1])


@functools.partial(
    pl.kernel,
    out_type=(
        jax.ShapeDtypeStruct(_ES, jnp.float32),
        jax.ShapeDtypeStruct(_ES, jnp.float32),
    ),
    mesh=_mesh,
    compiler_params=_params,
    scratch_types=[
        pltpu.VMEM((NCHUNK, CH), jnp.int32),
        pltpu.VMEM((NCHUNK, CH), jnp.int32),
        pltpu.VMEM((NCHUNK, CH), jnp.int32),
        pltpu.VMEM((NPF,), jnp.float32),
        pltpu.VMEM((NPF,), jnp.float32),
        pltpu.VMEM((NP,), jnp.float32),
        pltpu.VMEM((NCHUNK, CH), jnp.float32),
        pltpu.VMEM((NCHUNK, CH), jnp.float32),
    ],
)
def _sc_gate_a(dst_hbm, mid_hbm, et_hbm, s1_hbm, s2_hbm, d0_hbm,
               part_hbm, d0e_hbm,
               dstv, midv, etv, s1v, s2v, d0v, partv, d0ev):
    c = lax.axis_index("c")
    s = lax.axis_index("s")
    wid = c * NS + s
    pltpu.sync_copy(dst_hbm.at[wid], dstv)
    pltpu.sync_copy(mid_hbm.at[wid], midv)
    pltpu.sync_copy(et_hbm.at[wid], etv)
    pltpu.sync_copy(s1_hbm, s1v)
    pltpu.sync_copy(s2_hbm, s2v)
    pltpu.sync_copy(d0_hbm, d0v)

    def chunk(g, carry):
        def grp(i, carry2):
            sl = pl.ds(L * i, L)
            d16 = dstv[g, sl]
            m16 = midv[g, sl]
            base = etv[g, sl] * NP
            s1g = plsc.load_gather(s1v, [base + d16])
            s2g = plsc.load_gather(s2v, [base + m16])
            d0g = plsc.load_gather(d0v, [d16])
            partv[g, sl] = s1g + s2g
            d0ev[g, sl] = d0g
            return carry2

        lax.fori_loop(0, CH // L, grp, 0)
        return carry

    lax.fori_loop(0, NCHUNK, chunk, 0)
    pltpu.sync_copy(partv, part_hbm.at[wid])
    pltpu.sync_copy(d0ev, d0e_hbm.at[wid])


@functools.partial(
    pl.kernel,
    out_type=(
        jax.ShapeDtypeStruct(_ES, jnp.float32),
        jax.ShapeDtypeStruct(_ES, jnp.float32),
        jax.ShapeDtypeStruct(_ES, jnp.int32),
    ),
    mesh=_mesh,
    compiler_params=_params,
    scratch_types=[
        pltpu.VMEM((NCHUNK, CH), jnp.int32),
        pltpu.VMEM((NCHUNK, CH), jnp.int32),
        pltpu.VMEM((NCHUNK, CH), jnp.int32),
        pltpu.VMEM((NCHUNK, CH), jnp.float32),
        pltpu.VMEM((NCHUNK, CH), jnp.float32),
        pltpu.VMEM((NPF,), jnp.float32),
        pltpu.VMEM((NP,), jnp.float32),
        pltpu.VMEM((NP,), jnp.float32),
        pltpu.VMEM((NP,), jnp.float32),
        pltpu.VMEM((R * TW,), jnp.float32),
        pltpu.VMEM((NB,), jnp.float32),
        pltpu.VMEM((NCHUNK, CH), jnp.float32),
        pltpu.VMEM((NCHUNK, CH), jnp.float32),
        pltpu.VMEM((NCHUNK, CH), jnp.int32),
    ],
)
def _sc_gate_b(src_hbm, dst_hbm, et_hbm, part_hbm, d0e_hbm, s3_hbm, d2_hbm,
               lx_hbm, ly_hbm, t_hbm, b2_hbm,
               c1_hbm, sc_hbm, bi_hbm,
               srcv, dstv, etv, partv, d0ev, s3v, d2v, lxv, lyv, tv, b2v,
               c1v, scv, biv):
    c = lax.axis_index("c")
    s = lax.axis_index("s")
    wid = c * NS + s
    pltpu.sync_copy(src_hbm.at[wid], srcv)
    pltpu.sync_copy(dst_hbm.at[wid], dstv)
    pltpu.sync_copy(et_hbm.at[wid], etv)
    pltpu.sync_copy(part_hbm.at[wid], partv)
    pltpu.sync_copy(d0e_hbm.at[wid], d0ev)
    pltpu.sync_copy(s3_hbm, s3v)
    pltpu.sync_copy(d2_hbm, d2v)
    pltpu.sync_copy(lx_hbm, lxv)
    pltpu.sync_copy(ly_hbm, lyv)
    pltpu.sync_copy(t_hbm, tv)
    pltpu.sync_copy(b2_hbm, b2v)

    b2lo = b2v[pl.ds(0, L)]
    b2hi = b2v[pl.ds(L, L)]
    b2s = [b2lo[j] for j in range(L)] + [b2hi[j] for j in range(L)]

    def chunk(g, carry):
        def grp(i, carry2):
            sl = pl.ds(L * i, L)
            s16 = srcv[g, sl]
            d16 = dstv[g, sl]
            e16 = etv[g, sl]
            lxd = plsc.load_gather(lxv, [d16])
            lyd = plsc.load_gather(lyv, [d16])
            lxs = plsc.load_gather(lxv, [s16])
            lys = plsc.load_gather(lyv, [s16])
            dx = lxd - lxs
            dy = lyd - lys
            q = dx * dx + dy * dy + jnp.float32(1e-12)
            cnt = jnp.zeros((L,), jnp.int32)
            for j in range(NB):
                cnt = cnt + jnp.where(b2s[j] < q, 1, 0).astype(jnp.int32)
            tg = plsc.load_gather(tv, [e16 * TW + cnt])
            s3g = plsc.load_gather(s3v, [e16 * NP + s16])
            d2g = plsc.load_gather(d2v, [s16])
            x = partv[g, sl] + tg + s3g
            beta = 1.0 / (1.0 + jnp.exp(-x))
            scl = d0ev[g, sl] * d2g
            biv[g, sl] = cnt
            scv[g, sl] = scl
            c1v[g, sl] = scl * beta
            return carry2

        lax.fori_loop(0, CH // L, grp, 0)
        return carry

    lax.fori_loop(0, NCHUNK, chunk, 0)
    pltpu.sync_copy(c1v, c1_hbm.at[wid])
    pltpu.sync_copy(scv, sc_hbm.at[wid])
    pltpu.sync_copy(biv, bi_hbm.at[wid])


@functools.partial(
    pl.kernel,
    out_type=(
        jax.ShapeDtypeStruct(_ES, jnp.int32),
        jax.ShapeDtypeStruct(_ES, jnp.int32),
        jax.ShapeDtypeStruct(_ES, jnp.int32),
        jax.ShapeDtypeStruct(_ES, jnp.int32),
    ),
    mesh=_mesh,
    compiler_params=_params,
    scratch_types=[
        pltpu.VMEM((NCHUNK, CH), jnp.int32),
        pltpu.VMEM((NCHUNK, CH), jnp.int32),
        pltpu.VMEM((NCHUNK, CH), jnp.int32),
        pltpu.VMEM((NCHUNK, CH), jnp.int32),
        pltpu.VMEM((NCHUNK, CH), jnp.int32),
        pltpu.VMEM((NP,), jnp.float32),
        pltpu.VMEM((NP,), jnp.float32),
        pltpu.VMEM((NB,), jnp.float32),
        pltpu.VMEM((NCHUNK, CH), jnp.int32),
        pltpu.VMEM((NCHUNK, CH), jnp.int32),
        pltpu.VMEM((NCHUNK, CH), jnp.int32),
        pltpu.VMEM((NCHUNK, CH), jnp.int32),
    ],
)
def _sc_gate_c(src_hbm, mid_hbm, et_hbm, i0_hbm, i1_hbm, lx_hbm, ly_hbm,
               b2_hbm,
               gs_hbm, gm_hbm, b0_hbm, b1_hbm,
               srcv, midv, etv, i0v, i1v, lxv, lyv, b2v,
               gsv, gmv, b0v, b1v):
    c = lax.axis_index("c")
    s = lax.axis_index("s")
    wid = c * NS + s
    pltpu.sync_copy(src_hbm.at[wid], srcv)
    pltpu.sync_copy(mid_hbm.at[wid], midv)
    pltpu.sync_copy(et_hbm.at[wid], etv)
    pltpu.sync_copy(i0_hbm.at[wid], i0v)
    pltpu.sync_copy(i1_hbm.at[wid], i1v)
    pltpu.sync_copy(lx_hbm, lxv)
    pltpu.sync_copy(ly_hbm, lyv)
    pltpu.sync_copy(b2_hbm, b2v)

    b2lo = b2v[pl.ds(0, L)]
    b2hi = b2v[pl.ds(L, L)]
    b2s = [b2lo[j] for j in range(L)] + [b2hi[j] for j in range(L)]

    def chunk(g, carry):
        def grp(i, carry2):
            sl = pl.ds(L * i, L)
            base = etv[g, sl] * NP
            s16 = srcv[g, sl]
            gsv[g, sl] = base + s16
            gmv[g, sl] = base + midv[g, sl]
            lxs = plsc.load_gather(lxv, [s16])
            lys = plsc.load_gather(lyv, [s16])
            for iv, bv in ((i0v, b0v), (i1v, b1v)):
                n16 = iv[g, sl]
                lx = plsc.load_gather(lxv, [n16])
                ly = plsc.load_gather(lyv, [n16])
                dx = lxs - lx
                dy = lys - ly
                q = dx * dx + dy * dy + jnp.float32(1e-12)
                cnt = jnp.zeros((L,), jnp.int32)
                for j in range(NB):
                    cnt = cnt + jnp.where(b2s[j] < q, 1, 0).astype(jnp.int32)
                bv[g, sl] = cnt
            return carry2

        lax.fori_loop(0, CH // L, grp, 0)
        return carry

    lax.fori_loop(0, NCHUNK, chunk, 0)
    pltpu.sync_copy(gsv, gs_hbm.at[wid])
    pltpu.sync_copy(gmv, gm_hbm.at[wid])
    pltpu.sync_copy(b0v, b0_hbm.at[wid])
    pltpu.sync_copy(b1v, b1_hbm.at[wid])


@functools.partial(
    pl.kernel,
    out_type=jax.ShapeDtypeStruct((NC, NP, D), jnp.float32),
    mesh=_mesh,
    compiler_params=_params,
    scratch_types=(
        [pltpu.VMEM((CHR,), jnp.int32)] * 3
        + [pltpu.VMEM((CHR,), jnp.float32)] * 2
        + [pltpu.VMEM((CHR, D), jnp.float32)] * 2
    ) * 2 + [
        pltpu.VMEM((CHR, D), jnp.float32),
        pltpu.VMEM_SHARED((NP, D), jnp.float32),
        pltpu.SemaphoreType.DMA,
        pltpu.SemaphoreType.DMA,
        pltpu.SemaphoreType.DMA,
        pltpu.SemaphoreType.DMA,
        pltpu.SemaphoreType.DMA,
        pltpu.SemaphoreType.DMA,
    ],
)
def _sc_combine1(gs_hbm, gm_hbm, dst_hbm, c1_hbm, sc_hbm, h2_hbm,
                 zD_hbm, hp_hbm,
                 gsv0, gmv0, dstv0, c1v0, scv0, hsrc0, hmid0,
                 gsv1, gmv1, dstv1, c1v1, scv1, hsrc1, hmid1,
                 outr, acc, sem0, sem1, semL0, semL1, semM0, semM1):
    c = lax.axis_index("c")
    s = lax.axis_index("s")
    wid = c * NS + s
    pltpu.sync_copy(zD_hbm.at[pl.ds(s * NPS, NPS)], acc.at[pl.ds(s * NPS, NPS)])
    plsc.subcore_barrier()

    sets = ((gsv0, gmv0, dstv0, c1v0, scv0, hsrc0, hmid0, sem0, semL0, semM0),
            (gsv1, gmv1, dstv1, c1v1, scv1, hsrc1, hmid1, sem1, semL1, semM1))

    def _ld(g, st, op):
        gsv, gmv, dstv, c1v, scv = st[:5]
        sem = st[8]
        base = g * CHR
        op(gs_hbm.at[wid].at[pl.ds(base, CHR)], gsv, sem)
        op(gm_hbm.at[wid].at[pl.ds(base, CHR)], gmv, sem)
        op(dst_hbm.at[wid].at[pl.ds(base, CHR)], dstv, sem)
        op(c1_hbm.at[wid].at[pl.ds(base, CHR)], c1v, sem)
        op(sc_hbm.at[wid].at[pl.ds(base, CHR)], scv, sem)

    def fire_loads(g, st):
        _ld(g, st, pltpu.async_copy)

    def drain_loads(g, st):
        _ld(g, st, lambda a, b, sm: pltpu.make_async_copy(a, b, sm).wait())

    def fire(st):
        gsv, gmv = st[0], st[1]
        hsrc, hmid, sem, semm = st[5], st[6], st[7], st[9]
        pltpu.async_copy(h2_hbm.at[gsv], hsrc, sem)
        pltpu.async_copy(h2_hbm.at[gmv], hmid, semm)

    def drain(st):
        gsv, gmv = st[0], st[1]
        hsrc, hmid, sem, semm = st[5], st[6], st[7], st[9]
        pltpu.make_async_copy(h2_hbm.at[gsv], hsrc, sem).wait()
        pltpu.make_async_copy(h2_hbm.at[gmv], hmid, semm).wait()

    def work(g, st):
        dstv, c1v, scv, hsrc, hmid = st[2], st[3], st[4], st[5], st[6]

        def combine(i, carry2):
            sl = pl.ds(L * i, L)
            a16 = c1v[sl]
            b16 = scv[sl]
            for e in range(L):
                a = a16[e]
                b = b16[e]
                ee = L * i + e
                for v in range(D // L):
                    sv = pl.ds(L * v, L)
                    outr[ee, sv] = a * hsrc[ee, sv] + b * hmid[ee, sv]
            return carry2

        lax.fori_loop(0, CHR // L, combine, 0)
        pltpu.sync_copy(outr, acc.at[dstv], add=True)

    fire_loads(0, sets[0])
    drain_loads(0, sets[0])
    fire(sets[0])
    fire_loads(1, sets[1])

    def pair(p, carry):
        for b in (0, 1):
            g = 2 * p + b
            gn = jnp.minimum(g + 1, NCHR - 1)
            gnn = jnp.minimum(g + 2, NCHR - 1)
            drain_loads(gn, sets[1 - b])
            fire(sets[1 - b])
            drain(sets[b])
            work(g, sets[b])
            fire_loads(gnn, sets[b])
        return carry

    lax.fori_loop(0, NCHR // 2, pair, 0)
    drain(sets[0])
    drain_loads(0, sets[1])
    plsc.subcore_barrier()
    sl = pl.ds(s * NPS, NPS)
    pltpu.sync_copy(acc.at[sl], hp_hbm.at[c].at[sl])


@functools.partial(
    pl.kernel,
    out_type=(
        jax.ShapeDtypeStruct((NC, NP, D), jnp.float32),
        jax.ShapeDtypeStruct((NW, NP), jnp.float32),
    ),
    mesh=_mesh,
    compiler_params=_params,
    scratch_types=(
        [pltpu.VMEM((CHR,), jnp.int32)] * 3
        + [pltpu.VMEM((CHR,), jnp.float32)]
        + [pltpu.VMEM((CHR, D), jnp.float32)]
    ) * 2 + [
        pltpu.VMEM((TW, D), jnp.float32),
        pltpu.VMEM((NP,), jnp.float32),
        pltpu.VMEM((CHR, D), jnp.float32),
        pltpu.VMEM((CHR, D), jnp.float32),
        pltpu.VMEM_SHARED((NP, D), jnp.float32),
        pltpu.SemaphoreType.DMA,
        pltpu.SemaphoreType.DMA,
        pltpu.SemaphoreType.DMA,
        pltpu.SemaphoreType.DMA,
        pltpu.SemaphoreType.DMA,
        pltpu.SemaphoreType.DMA,
    ],
)
def _sc_stage2a(src_hbm, dst_hbm, bi_hbm, sc_hbm, f2_hbm, tg_hbm, zD_hbm,
                s1p_hbm, scp_hbm,
                srcv0, dstv0, biv0, scv0, f2r0,
                srcv1, dstv1, biv1, scv1, f2r1,
                tgv, hist, outr0, outr1, acc, sem0, sem1, semL0, semL1, semS0, semS1):
    c = lax.axis_index("c")
    s = lax.axis_index("s")
    wid = c * NS + s
    pltpu.sync_copy(tg_hbm, tgv)
    _zero1d(hist, NP)
    pltpu.sync_copy(zD_hbm.at[pl.ds(s * NPS, NPS)], acc.at[pl.ds(s * NPS, NPS)])
    plsc.subcore_barrier()

    sets = ((srcv0, dstv0, biv0, scv0, f2r0, sem0, semL0, outr0, semS0),
            (srcv1, dstv1, biv1, scv1, f2r1, sem1, semL1, outr1, semS1))

    def _ld(g, st, op):
        srcv, dstv, biv, scv = st[:4]
        sem = st[6]
        base = g * CHR
        op(src_hbm.at[wid].at[pl.ds(base, CHR)], srcv, sem)
        op(dst_hbm.at[wid].at[pl.ds(base, CHR)], dstv, sem)
        op(bi_hbm.at[wid].at[pl.ds(base, CHR)], biv, sem)
        op(sc_hbm.at[wid].at[pl.ds(base, CHR)], scv, sem)

    def fire_loads(g, st):
        _ld(g, st, pltpu.async_copy)

    def drain_loads(g, st):
        _ld(g, st, lambda a, b, sm: pltpu.make_async_copy(a, b, sm).wait())

    def fire(st):
        pltpu.async_copy(f2_hbm.at[st[0]], st[4], st[5])

    def drain(st):
        pltpu.make_async_copy(f2_hbm.at[st[0]], st[4], st[5]).wait()

    def work(g, st):
        dstv, biv, scv, f2r = st[1], st[2], st[3], st[4]
        outr, semS = st[7], st[8]
        pltpu.make_async_copy(outr, acc.at[dstv], semS).wait()

        def grp(i, carry2):
            sl = pl.ds(L * i, L)
            plsc.addupdate_scatter(hist, [dstv[sl]], scv[sl])
            b16 = biv[sl]
            s16 = scv[sl]
            for e in range(L):
                b = b16[e]
                scl = s16[e]
                ee = L * i + e
                for v in range(D // L):
                    sv = pl.ds(L * v, L)
                    outr[ee, sv] = scl * tgv[b, sv] * f2r[ee, sv]
            return carry2

        lax.fori_loop(0, CHR // L, grp, 0)
        pltpu.async_copy(outr, acc.at[dstv], semS, add=True)

    fire_loads(0, sets[0])
    drain_loads(0, sets[0])
    fire(sets[0])
    fire_loads(1, sets[1])
    _zero_rows(outr0)
    _zero_rows(outr1)
    pltpu.async_copy(outr0, acc.at[dstv0], semS0, add=True)
    pltpu.async_copy(outr1, acc.at[dstv0], semS1, add=True)

    def pair(p, carry):
        for b in (0, 1):
            g = 2 * p + b
            gn = jnp.minimum(g + 1, NCHR - 1)
            gnn = jnp.minimum(g + 2, NCHR - 1)
            drain_loads(gn, sets[1 - b])
            fire(sets[1 - b])
            drain(sets[b])
            work(g, sets[b])
            fire_loads(gnn, sets[b])
        return carry

    lax.fori_loop(0, NCHR // 2, pair, 0)
    drain(sets[0])
    drain_loads(0, sets[1])
    pltpu.make_async_copy(outr0, acc.at[dstv0], semS0).wait()
    pltpu.make_async_copy(outr1, acc.at[dstv1], semS1).wait()
    plsc.subcore_barrier()
    sl = pl.ds(s * NPS, NPS)
    pltpu.sync_copy(acc.at[sl], s1p_hbm.at[c].at[sl])
    pltpu.sync_copy(hist, scp_hbm.at[wid])


@functools.partial(
    pl.kernel,
    out_type=jax.ShapeDtypeStruct((NC, NP, D), jnp.float32),
    mesh=_mesh,
    compiler_params=_params,
    scratch_types=(
        [pltpu.VMEM((CHR,), jnp.int32)] * 5
        + [pltpu.VMEM((CHR,), jnp.float32)]
        + [pltpu.VMEM((CHR, D), jnp.float32)] * 2
    ) * 2 + [
        pltpu.VMEM((TW, D), jnp.float32),
        pltpu.VMEM((CHR, D), jnp.float32),
        pltpu.VMEM_SHARED((NP, D), jnp.float32),
        pltpu.SemaphoreType.DMA,
        pltpu.SemaphoreType.DMA,
        pltpu.SemaphoreType.DMA,
        pltpu.SemaphoreType.DMA,
        pltpu.SemaphoreType.DMA,
        pltpu.SemaphoreType.DMA,
    ],
)
def _sc_stage2b(i0_hbm, i1_hbm, dst_hbm, b0_hbm, b1_hbm, sc_hbm, f2_hbm,
                tg_hbm, zD_hbm,
                s2p_hbm,
                i0v0, i1v0, dstv0, b0v0, b1v0, scv0, f0r0, f1r0,
                i0v1, i1v1, dstv1, b0v1, b1v1, scv1, f0r1, f1r1,
                tgv, outr, acc, sem0, sem1, semL0, semL1, semM0, semM1):
    c = lax.axis_index("c")
    s = lax.axis_index("s")
    wid = c * NS + s
    pltpu.sync_copy(tg_hbm, tgv)
    pltpu.sync_copy(zD_hbm.at[pl.ds(s * NPS, NPS)], acc.at[pl.ds(s * NPS, NPS)])
    plsc.subcore_barrier()

    sets = ((i0v0, i1v0, dstv0, b0v0, b1v0, scv0, f0r0, f1r0, sem0, semL0, semM0),
            (i0v1, i1v1, dstv1, b0v1, b1v1, scv1, f0r1, f1r1, sem1, semL1, semM1))

    def _ld(g, st, op):
        i0v, i1v, dstv, b0v, b1v, scv = st[:6]
        sem = st[9]
        base = g * CHR
        op(i0_hbm.at[wid].at[pl.ds(base, CHR)], i0v, sem)
        op(i1_hbm.at[wid].at[pl.ds(base, CHR)], i1v, sem)
        op(dst_hbm.at[wid].at[pl.ds(base, CHR)], dstv, sem)
        op(b0_hbm.at[wid].at[pl.ds(base, CHR)], b0v, sem)
        op(b1_hbm.at[wid].at[pl.ds(base, CHR)], b1v, sem)
        op(sc_hbm.at[wid].at[pl.ds(base, CHR)], scv, sem)

    def fire_loads(g, st):
        _ld(g, st, pltpu.async_copy)

    def drain_loads(g, st):
        _ld(g, st, lambda a, b, sm: pltpu.make_async_copy(a, b, sm).wait())

    def fire(st):
        pltpu.async_copy(f2_hbm.at[st[0]], st[6], st[8])
        pltpu.async_copy(f2_hbm.at[st[1]], st[7], st[10])

    def drain(st):
        pltpu.make_async_copy(f2_hbm.at[st[0]], st[6], st[8]).wait()
        pltpu.make_async_copy(f2_hbm.at[st[1]], st[7], st[10]).wait()

    def work(g, st):
        dstv, b0v, b1v, scv, f0r, f1r = (st[2], st[3], st[4], st[5],
                                         st[6], st[7])

        def combine(i, carry2):
            sl = pl.ds(L * i, L)
            b016 = b0v[sl]
            b116 = b1v[sl]
            s16 = scv[sl]
            for e in range(L):
                b0 = b016[e]
                b1 = b116[e]
                scl = jnp.float32(0.5) * s16[e]
                ee = L * i + e
                for v in range(D // L):
                    sv = pl.ds(L * v, L)
                    outr[ee, sv] = scl * (tgv[b0, sv] * f0r[ee, sv]
                                          + tgv[b1, sv] * f1r[ee, sv])
            return carry2

        lax.fori_loop(0, CHR // L, combine, 0)
        pltpu.sync_copy(outr, acc.at[dstv], add=True)

    fire_loads(0, sets[0])
    drain_loads(0, sets[0])
    fire(sets[0])
    fire_loads(1, sets[1])

    def pair(p, carry):
        for b in (0, 1):
            g = 2 * p + b
            gn = jnp.minimum(g + 1, NCHR - 1)
            gnn = jnp.minimum(g + 2, NCHR - 1)
            drain_loads(gn, sets[1 - b])
            fire(sets[1 - b])
            drain(sets[b])
            work(g, sets[b])
            fire_loads(gnn, sets[b])
        return carry

    lax.fori_loop(0, NCHR // 2, pair, 0)
    drain(sets[0])
    drain_loads(0, sets[1])
    plsc.subcore_barrier()
    sl = pl.ds(s * NPS, NPS)
    pltpu.sync_copy(acc.at[sl], s2p_hbm.at[c].at[sl])


def _tc_prep_body(feat_ref, fc2_ref, a8_ref, degp_ref,
                  h2_ref, s1_ref, s2_ref, s3_ref, d0_ref, d2_ref):
    h2 = jnp.dot(feat_ref[...], fc2_ref[0], preferred_element_type=jnp.float32)
    h2_ref[...] = h2
    sc3 = jnp.dot(h2, a8_ref[0], preferred_element_type=jnp.float32)
    s1_ref[...] = sc3[:, 0].reshape(BM // D, D)
    s2_ref[...] = sc3[:, 1].reshape(BM // D, D)
    s3_ref[...] = sc3[:, 2].reshape(BM // D, D)
    degsum = jnp.sum(degp_ref[...], axis=0)
    d0_ref[...] = lax.rsqrt(jnp.maximum(degsum[0], 1.0)).reshape(BM // D, D)
    d2_ref[...] = lax.rsqrt(jnp.maximum(degsum[1], 1.0)).reshape(BM // D, D)


def _tc_prep(featp, fc2, a8, degp):
    nblk = NP // BM
    rb = BM // D
    return pl.pallas_call(
        _tc_prep_body,
        grid=(R, nblk),
        in_specs=[
            pl.BlockSpec((BM, D), lambda r, j: (j, 0)),
            pl.BlockSpec((1, D, D), lambda r, j: (r, 0, 0)),
            pl.BlockSpec((1, D, 8), lambda r, j: (r, 0, 0)),
            pl.BlockSpec((NW, 2, BM), lambda r, j: (0, 0, j)),
        ],
        out_specs=[
            pl.BlockSpec((BM, D), lambda r, j: (r * (NP // BM) + j, 0)),
            pl.BlockSpec((rb, D), lambda r, j: (r * (NP // BM) + j, 0)),
            pl.BlockSpec((rb, D), lambda r, j: (r * (NP // BM) + j, 0)),
            pl.BlockSpec((rb, D), lambda r, j: (r * (NP // BM) + j, 0)),
            pl.BlockSpec((rb, D), lambda r, j: (j, 0)),
            pl.BlockSpec((rb, D), lambda r, j: (j, 0)),
        ],
        out_shape=[
            jax.ShapeDtypeStruct((NPF, D), jnp.float32),
            jax.ShapeDtypeStruct((NPF // D, D), jnp.float32),
            jax.ShapeDtypeStruct((NPF // D, D), jnp.float32),
            jax.ShapeDtypeStruct((NPF // D, D), jnp.float32),
            jax.ShapeDtypeStruct((NP // D, D), jnp.float32),
            jax.ShapeDtypeStruct((NP // D, D), jnp.float32),
        ],
    )(featp, fc2, a8, degp)


def _tc_mid_body(hp_ref, fcp_ref, f2_ref):
    h = jnp.tanh(hp_ref[0] + hp_ref[1])
    f2_ref[...] = jnp.dot(h, fcp_ref[...], preferred_element_type=jnp.float32)


def _tc_mid(hpart, fc_p):
    return pl.pallas_call(
        _tc_mid_body,
        grid=(NP // BM,),
        in_specs=[
            pl.BlockSpec((NC, BM, D), lambda j: (0, j, 0)),
            pl.BlockSpec((D, D), lambda j: (0, 0)),
        ],
        out_specs=pl.BlockSpec((BM, D), lambda j: (j, 0)),
        out_shape=jax.ShapeDtypeStruct((NP, D), jnp.float32),
    )(hpart, fc_p)


def _tc_final_body(s1p_ref, s2p_ref, scp_ref, wa_ref, wb_ref, bb_ref, out_ref):
    s1 = s1p_ref[0] + s1p_ref[1]
    s2 = s2p_ref[0] + s2p_ref[1]
    sc = jnp.sum(scp_ref[...], axis=0)
    rst = (jnp.dot(s1, wa_ref[...], preferred_element_type=jnp.float32)
           + jnp.dot(s2, wb_ref[...], preferred_element_type=jnp.float32)
           + sc[:, None] * bb_ref[0][None, :])
    out_ref[...] = jnp.tanh(rst)


def _tc_final(s1p, s2p, scp, wa, wb, bb):
    return pl.pallas_call(
        _tc_final_body,
        grid=(NP // BM,),
        in_specs=[
            pl.BlockSpec((NC, BM, D), lambda j: (0, j, 0)),
            pl.BlockSpec((NC, BM, D), lambda j: (0, j, 0)),
            pl.BlockSpec((NW, BM), lambda j: (0, j)),
            pl.BlockSpec((D, D), lambda j: (0, 0)),
            pl.BlockSpec((D, D), lambda j: (0, 0)),
            pl.BlockSpec((1, D), lambda j: (0, 0)),
        ],
        out_specs=pl.BlockSpec((BM, D), lambda j: (j, 0)),
        out_shape=jax.ShapeDtypeStruct((NP, D), jnp.float32),
    )(s1p, s2p, scp, wa, wb, bb)


def kernel(feat, loc, src, dst, mid, inter_ids, etype, boundaries, embed_table,
           fc2, fcd, fc_w1, fc_w2, vec_a, fc_p, agg_W, agg_b, G):
    f32 = jnp.float32
    i32 = jnp.int32

    featp = jnp.zeros((NP, D), f32).at[:N].set(feat)
    locx = jnp.zeros((NP,), f32).at[:N].set(loc[:, 0])
    locy = jnp.zeros((NP,), f32).at[:N].set(loc[:, 1])

    pad = EP - E

    def pad_idx(x, val):
        return jnp.concatenate(
            [x.astype(i32), jnp.full((pad,), val, i32)]).reshape(NW, NCHUNK, CH)

    src_r = pad_idx(src, DUMMY)
    dst_r = pad_idx(dst, DUMMY)
    mid_r = pad_idx(mid, DUMMY)
    et_r = pad_idx(etype, 0)
    i0_r = pad_idx(inter_ids[:, 0], DUMMY)
    i1_r = pad_idx(inter_ids[:, 1], DUMMY)

    b2 = (boundaries * boundaries).astype(f32)

    a1 = jnp.einsum('rdk,rko->rd', fc_w1[:, :D, :], vec_a)
    a2 = jnp.einsum('rdk,rko->rd', fc_w2[:, :D, :], vec_a)
    a3 = jnp.einsum('rdk,rko->rd', fc_w2[:, D:, :], vec_a)
    a8 = jnp.concatenate(
        [a1[..., None], a2[..., None], a3[..., None],
         jnp.zeros((R, D, 5), f32)], axis=-1)
    td = jnp.einsum('rdk,rko->rd', fc_w1[:, D:, :], vec_a)
    tvec = jnp.einsum('red,rd->re', fcd, td)
    tbl = jnp.einsum('be,re->rb', embed_table, tvec)
    tflat = jnp.zeros((R, TW), f32).at[:, :NB + 1].set(tbl).reshape(-1)
    tgp = jnp.zeros((TW, D), f32).at[:NB + 1].set(embed_table @ G)

    zD = jnp.zeros((NP, D), f32)

    degp = _sc_degrees(dst_r, src_r)
    h2flat, s1t, s2t, s3t, d0t, d2t = _tc_prep(featp, fc2, a8, degp)
    s1f = s1t.reshape(-1)
    s2f = s2t.reshape(-1)
    s3f = s3t.reshape(-1)
    d0f = d0t.reshape(-1)
    d2f = d2t.reshape(-1)
    part, d0e = _sc_gate_a(dst_r, mid_r, et_r, s1f, s2f, d0f)
    c1_r, sc_r, bi_r = _sc_gate_b(src_r, dst_r, et_r, part, d0e, s3f, d2f,
                                  locx, locy, tflat, b2)
    gs_r, gm_r, b0_r, b1_r = _sc_gate_c(src_r, mid_r, et_r, i0_r, i1_r,
                                        locx, locy, b2)

    def rr(x):
        return x.reshape(NW, ET)

    hpart = _sc_combine1(rr(gs_r), rr(gm_r), rr(dst_r), rr(c1_r), rr(sc_r),
                         h2flat, zD)
    f2 = _tc_mid(hpart, fc_p)
    s1p, scp = _sc_stage2a(rr(src_r), rr(dst_r), rr(bi_r), rr(sc_r), f2, tgp,
                           zD)
    s2p = _sc_stage2b(rr(i0_r), rr(i1_r), rr(dst_r), rr(b0_r), rr(b1_r),
                      rr(sc_r), f2, tgp, zD)
    outp = _tc_final(s1p, s2p, scp, agg_W[:D], agg_W[D:],
                     agg_b.reshape(1, D).astype(f32))
    return outp[:N]
